# Initial kernel scaffold; baseline (speedup 1.0000x reference)
import jax
import jax.numpy as jnp
from jax.experimental import pallas as pl


def kernel(n_id, memory, last_update):
    raise NotImplementedError("write your pallas kernel here")


# speedup vs baseline: 1.7466x; 1.7466x over previous
import functools

import jax
import jax.numpy as jnp
from jax import lax
from jax.experimental import pallas as pl
from jax.experimental.pallas import tpu as pltpu
from jax.experimental.pallas import tpu_sc as plsc

NUM_CORES = 2
NUM_SUBCORES = 16
NW = NUM_CORES * NUM_SUBCORES
CHUNK = 128


def _gather_body(n_chunks, n_id_hbm, memory_hbm, last_hbm,
                 mem_out_hbm, last_out_hbm,
                 idx_v, rows_v, last_v, sem_idx, sem_rows, sem_last):
    wid = lax.axis_index("s") * NUM_CORES + lax.axis_index("c")
    b_per_w = n_chunks * CHUNK
    base = wid * b_per_w

    for j in range(n_chunks):
        pltpu.async_copy(
            n_id_hbm.at[pl.ds(base + j * CHUNK, CHUNK)], idx_v.at[j],
            sem_idx).wait()
        pltpu.async_copy(
            memory_hbm.at[idx_v.at[j]],
            rows_v.at[pl.ds(j * CHUNK, CHUNK), :], sem_rows)
        pltpu.async_copy(
            last_hbm.at[idx_v.at[j]],
            last_v.at[pl.ds(j * CHUNK, CHUNK)], sem_last)

    for j in range(n_chunks):
        pltpu.make_async_copy(
            memory_hbm.at[idx_v.at[j]],
            rows_v.at[pl.ds(j * CHUNK, CHUNK), :], sem_rows).wait()
        pltpu.make_async_copy(
            last_hbm.at[idx_v.at[j]],
            last_v.at[pl.ds(j * CHUNK, CHUNK)], sem_last).wait()

    pltpu.sync_copy(rows_v, mem_out_hbm.at[pl.ds(base, b_per_w), :])
    pltpu.sync_copy(last_v, last_out_hbm.at[pl.ds(base, b_per_w)])


@jax.jit
def kernel(n_id, memory, last_update):
    B = n_id.shape[0]
    D = memory.shape[1]
    assert B % (NW * CHUNK) == 0
    n_chunks = B // (NW * CHUNK)
    b_per_w = n_chunks * CHUNK

    mesh = plsc.VectorSubcoreMesh(core_axis_name="c", subcore_axis_name="s")
    run = pl.kernel(
        functools.partial(_gather_body, n_chunks),
        out_type=(
            jax.ShapeDtypeStruct((B, D), memory.dtype),
            jax.ShapeDtypeStruct((B,), last_update.dtype),
        ),
        mesh=mesh,
        scratch_types=[
            pltpu.VMEM((n_chunks, CHUNK), jnp.int32),
            pltpu.VMEM((b_per_w, D), jnp.float32),
            pltpu.VMEM((b_per_w,), jnp.int32),
            pltpu.SemaphoreType.DMA,
            pltpu.SemaphoreType.DMA,
            pltpu.SemaphoreType.DMA,
        ],
    )
    return run(n_id, memory, last_update)


# speedup vs baseline: 1.7549x; 1.0048x over previous
import functools

import jax
import jax.numpy as jnp
from jax import lax
from jax.experimental import pallas as pl
from jax.experimental.pallas import tpu as pltpu
from jax.experimental.pallas import tpu_sc as plsc

NUM_CORES = 2
NUM_SUBCORES = 16
NW = NUM_CORES * NUM_SUBCORES
CHUNK = 128


def _gather_body(n_chunks, n_id_hbm, memory_hbm, last_hbm,
                 mem_out_hbm, last_out_hbm,
                 idx_v, rows_v, last_v, *sems):
    sem_idx, sem_last, sem_store = sems[0], sems[1], sems[2]
    chunk_sems = sems[3:]
    wid = lax.axis_index("s") * NUM_CORES + lax.axis_index("c")
    b_per_w = n_chunks * CHUNK
    base = wid * b_per_w

    pltpu.async_copy(n_id_hbm.at[pl.ds(base, b_per_w)], idx_v, sem_idx).wait()

    for j in range(n_chunks):
        idx_j = idx_v.at[pl.ds(j * CHUNK, CHUNK)]
        pltpu.async_copy(
            memory_hbm.at[idx_j],
            rows_v.at[pl.ds(j * CHUNK, CHUNK), :], chunk_sems[j])
        pltpu.async_copy(
            last_hbm.at[idx_j],
            last_v.at[pl.ds(j * CHUNK, CHUNK)], sem_last)

    for j in range(n_chunks):
        pltpu.make_async_copy(
            memory_hbm.at[idx_v.at[pl.ds(j * CHUNK, CHUNK)]],
            rows_v.at[pl.ds(j * CHUNK, CHUNK), :], chunk_sems[j]).wait()
        pltpu.async_copy(
            rows_v.at[pl.ds(j * CHUNK, CHUNK), :],
            mem_out_hbm.at[pl.ds(base + j * CHUNK, CHUNK), :], sem_store)

    for j in range(n_chunks):
        pltpu.make_async_copy(
            last_hbm.at[idx_v.at[pl.ds(j * CHUNK, CHUNK)]],
            last_v.at[pl.ds(j * CHUNK, CHUNK)], sem_last).wait()
    pltpu.async_copy(last_v, last_out_hbm.at[pl.ds(base, b_per_w)], sem_store)

    for j in range(n_chunks):
        pltpu.make_async_copy(
            rows_v.at[pl.ds(j * CHUNK, CHUNK), :],
            mem_out_hbm.at[pl.ds(base + j * CHUNK, CHUNK), :],
            sem_store).wait()
    pltpu.make_async_copy(
        last_v, last_out_hbm.at[pl.ds(base, b_per_w)], sem_store).wait()


@jax.jit
def kernel(n_id, memory, last_update):
    B = n_id.shape[0]
    D = memory.shape[1]
    assert B % (NW * CHUNK) == 0
    n_chunks = B // (NW * CHUNK)
    b_per_w = n_chunks * CHUNK

    mesh = plsc.VectorSubcoreMesh(core_axis_name="c", subcore_axis_name="s")
    run = pl.kernel(
        functools.partial(_gather_body, n_chunks),
        out_type=(
            jax.ShapeDtypeStruct((B, D), memory.dtype),
            jax.ShapeDtypeStruct((B,), last_update.dtype),
        ),
        mesh=mesh,
        scratch_types=[
            pltpu.VMEM((b_per_w,), jnp.int32),
            pltpu.VMEM((b_per_w, D), jnp.float32),
            pltpu.VMEM((b_per_w,), jnp.int32),
            pltpu.SemaphoreType.DMA,
            pltpu.SemaphoreType.DMA,
            pltpu.SemaphoreType.DMA,
        ] + [pltpu.SemaphoreType.DMA] * n_chunks,
    )
    return run(n_id, memory, last_update)
